# baseline (speedup 1.0000x reference)
import jax
import jax.numpy as jnp
from jax import lax
from jax.experimental import pallas as pl
from jax.experimental.pallas import tpu as pltpu
from jax.experimental.pallas import tpu_sc as plsc

B, S = 4096, 200
WORD_SIZE, POS_SIZE = 64, 16
OUT_SIZE = WORD_SIZE + 2 * POS_SIZE
WPAD = 128
N = B * S

NC, NS = 2, 16
NW = NC * NS
GRP = 128
NGRP = S
SLAB = 5
NBLK = NGRP // (8 * SLAB)


def _splat(v):
    return jnp.full((16,), v, jnp.int32)


def _emb_body(wi_hbm, p1i_hbm, p2i_hbm, wt_hbm, p1t_hbm, p2t_hbm, out_hbm,
              idxw_v, idx1_v, idx2_v, wbuf, tbuf, post1, post2,
              semG0, semG1, semW0, semW1):
    semG = (semG0, semG1)
    semW = (semW0, semW1)
    wid = lax.axis_index("s") * NC + lax.axis_index("c")
    iota16 = lax.iota(jnp.int32, 16)

    pltpu.sync_copy(p1t_hbm, post1)
    pltpu.sync_copy(p2t_hbm, post2)

    def fire_G(b, sbl, si):
        pltpu.async_copy(wt_hbm.at[idxw_v.at[sbl, si]], wbuf.at[b], semG[b])

    def wait_G(b):
        pltpu.make_async_copy(wt_hbm.at[idxw_v.at[0, 0]], wbuf.at[b],
                              semG[b]).wait()

    def fire_W(s, b):
        pltpu.async_copy(tbuf.at[b], out_hbm.at[s, :, wid], semW[b])

    def wait_W(b):
        pltpu.make_async_copy(tbuf.at[b], out_hbm.at[0, :, wid],
                              semW[b]).wait()

    def transpose_group(b, sbl, si):
        tb = tbuf.at[b]
        wb = wbuf.at[b]

        @pl.loop(0, 8)
        def _(t):
            off = t * 16
            bv = iota16 + off
            p1 = idx1_v[sbl, si, pl.ds(off, 16)]
            p2 = idx2_v[sbl, si, pl.ds(off, 16)]
            ph1 = lax.shift_right_logical(p1, 7)
            pl1 = lax.bitwise_and(p1, 127)
            ph2 = lax.shift_right_logical(p2, 7)
            pl2 = lax.bitwise_and(p2, 127)
            for c in range(WORD_SIZE):
                v = plsc.load_gather(wb, [bv, _splat(c)])
                tb[c // 8, c % 8, pl.ds(off, 16)] = v
            for c in range(POS_SIZE):
                v = plsc.load_gather(post1, [_splat(c // 8), ph1,
                                             _splat(c % 8), pl1])
                tb[8 + c // 8, c % 8, pl.ds(off, 16)] = v
            for c in range(POS_SIZE):
                v = plsc.load_gather(post2, [_splat(c // 8), ph2,
                                             _splat(c % 8), pl2])
                tb[10 + c // 8, c % 8, pl.ds(off, 16)] = v

    def step(k, gl, b, fire_next):
        wait_G(b)
        if fire_next:
            gl1 = gl + 1
            fire_G(1 - b, gl1 >> 3, gl1 & 7)
        g_total = 40 * k + gl

        @pl.when(g_total >= 2)
        def _():
            wait_W(b)

        transpose_group(b, gl >> 3, gl & 7)
        fire_W(g_total, b)

    @pl.loop(0, NBLK)
    def _(k):
        rows = pl.ds(SLAB * k, SLAB)
        pltpu.sync_copy(wi_hbm.at[rows, wid], idxw_v)
        pltpu.sync_copy(p1i_hbm.at[rows, wid], idx1_v)
        pltpu.sync_copy(p2i_hbm.at[rows, wid], idx2_v)
        fire_G(0, 0, 0)

        @pl.loop(0, 19)
        def _(j):
            step(k, 2 * j, 0, True)
            step(k, 2 * j + 1, 1, True)

        step(k, 38, 0, True)
        step(k, 39, 1, False)

    wait_W(0)
    wait_W(1)


@jax.jit
def _embed(word, pos1, pos2, word_table, pos1_table, pos2_table):
    mesh = plsc.VectorSubcoreMesh(core_axis_name="c", subcore_axis_name="s")
    k = pl.kernel(
        _emb_body,
        out_type=jax.ShapeDtypeStruct((S, 12, 32, 8, 128), jnp.float32),
        mesh=mesh,
        compiler_params=pltpu.CompilerParams(needs_layout_passes=False),
        scratch_types=[
            pltpu.VMEM((SLAB, 8, 128), jnp.int32),
            pltpu.VMEM((SLAB, 8, 128), jnp.int32),
            pltpu.VMEM((SLAB, 8, 128), jnp.int32),
            pltpu.VMEM((2, GRP, WPAD), jnp.float32),
            pltpu.VMEM((2, 12, 8, 128), jnp.float32),
            pltpu.VMEM((2, 8, 8, 128), jnp.float32),
            pltpu.VMEM((2, 8, 8, 128), jnp.float32),
            pltpu.SemaphoreType.DMA,
            pltpu.SemaphoreType.DMA,
            pltpu.SemaphoreType.DMA,
            pltpu.SemaphoreType.DMA,
        ],
    )
    wi = word.T.reshape(25, 8, 32, 128).transpose(0, 2, 1, 3)
    p1i = pos1.T.reshape(25, 8, 32, 128).transpose(0, 2, 1, 3)
    p2i = pos2.T.reshape(25, 8, 32, 128).transpose(0, 2, 1, 3)
    wt = jnp.pad(word_table, ((0, 0), (0, WPAD - WORD_SIZE)))
    p1t = jnp.pad(pos1_table.T, ((0, 0), (0, 24))).reshape(
        2, 8, 8, 128).transpose(0, 2, 1, 3)
    p2t = jnp.pad(pos2_table.T, ((0, 0), (0, 24))).reshape(
        2, 8, 8, 128).transpose(0, 2, 1, 3)
    out5 = k(wi, p1i, p2i, wt, p1t, p2t)
    return out5.transpose(2, 4, 0, 1, 3).reshape(B, S, OUT_SIZE)


def kernel(word, pos1, pos2, chars, word_table, pos1_table, pos2_table):
    del chars
    return _embed(word, pos1, pos2, word_table, pos1_table, pos2_table)


# speedup vs baseline: 1.6829x; 1.6829x over previous
import jax
import jax.numpy as jnp
from jax import lax
from jax.experimental import pallas as pl
from jax.experimental.pallas import tpu as pltpu
from jax.experimental.pallas import tpu_sc as plsc

B, S = 4096, 200
WORD_SIZE, POS_SIZE = 64, 16
OUT_SIZE = WORD_SIZE + 2 * POS_SIZE
WPAD = 128
N = B * S

NC, NS = 2, 16
NW = NC * NS
GRP = 128
NGRP = S
SLAB = 5
NBLK = NGRP // (8 * SLAB)


def _splat(v):
    return jnp.full((16,), v, jnp.int32)


def _emb_body(wi_hbm, p1i_hbm, p2i_hbm, wt_hbm, p1t_hbm, p2t_hbm, out_hbm,
              idxw_v, idx1_v, idx2_v, wbuf, tbuf, post1, post2,
              semG0, semG1, semW0, semW1):
    semG = (semG0, semG1)
    semW = (semW0, semW1)
    wid = lax.axis_index("s") * NC + lax.axis_index("c")
    iota16 = lax.iota(jnp.int32, 16)

    pltpu.sync_copy(p1t_hbm, post1)
    pltpu.sync_copy(p2t_hbm, post2)

    def fire_G(b, sbl, si):
        pltpu.async_copy(wt_hbm.at[idxw_v.at[sbl, si]], wbuf.at[b], semG[b])

    def wait_G(b):
        pltpu.make_async_copy(wt_hbm.at[idxw_v.at[0, 0]], wbuf.at[b],
                              semG[b]).wait()

    def fire_W(s, b):
        pltpu.async_copy(tbuf.at[b], out_hbm.at[s, :, wid], semW[b])

    def wait_W(b):
        pltpu.make_async_copy(tbuf.at[b], out_hbm.at[0, :, wid],
                              semW[b]).wait()

    def transpose_group(b, sbl, si):
        tb = tbuf.at[b]
        wb = wbuf.at[b]

        colv = [iota16 + 16 * c4 for c4 in range(4)]
        chi = [lax.shift_right_logical(cv, 3) for cv in colv]
        clo = [lax.bitwise_and(cv, 7) for cv in colv]

        @pl.loop(0, 16)
        def _(j):
            rb = lax.bitwise_and(iota16 + j, 15)
            for t in range(8):
                rowv = rb + 16 * t
                for c4 in range(4):
                    v = plsc.load_gather(wb, [rowv, colv[c4]])
                    plsc.store_scatter(tb, [chi[c4], clo[c4], rowv], v)

        @pl.loop(0, 8)
        def _(t):
            off = t * 16
            p1 = idx1_v[sbl, si, pl.ds(off, 16)]
            p2 = idx2_v[sbl, si, pl.ds(off, 16)]
            ph1 = lax.shift_right_logical(p1, 7)
            pl1 = lax.bitwise_and(p1, 127)
            ph2 = lax.shift_right_logical(p2, 7)
            pl2 = lax.bitwise_and(p2, 127)
            for c in range(POS_SIZE):
                v = plsc.load_gather(post1, [_splat(c // 8), ph1,
                                             _splat(c % 8), pl1])
                tb[8 + c // 8, c % 8, pl.ds(off, 16)] = v
            for c in range(POS_SIZE):
                v = plsc.load_gather(post2, [_splat(c // 8), ph2,
                                             _splat(c % 8), pl2])
                tb[10 + c // 8, c % 8, pl.ds(off, 16)] = v

    def step(k, gl, b, fire_next):
        wait_G(b)
        if fire_next:
            gl1 = gl + 1
            fire_G(1 - b, gl1 >> 3, gl1 & 7)
        g_total = 40 * k + gl

        @pl.when(g_total >= 2)
        def _():
            wait_W(b)

        transpose_group(b, gl >> 3, gl & 7)
        fire_W(g_total, b)

    @pl.loop(0, NBLK)
    def _(k):
        rows = pl.ds(SLAB * k, SLAB)
        pltpu.sync_copy(wi_hbm.at[rows, wid], idxw_v)
        pltpu.sync_copy(p1i_hbm.at[rows, wid], idx1_v)
        pltpu.sync_copy(p2i_hbm.at[rows, wid], idx2_v)
        fire_G(0, 0, 0)

        @pl.loop(0, 19)
        def _(j):
            step(k, 2 * j, 0, True)
            step(k, 2 * j + 1, 1, True)

        step(k, 38, 0, True)
        step(k, 39, 1, False)

    wait_W(0)
    wait_W(1)


@jax.jit
def _embed(word, pos1, pos2, word_table, pos1_table, pos2_table):
    mesh = plsc.VectorSubcoreMesh(core_axis_name="c", subcore_axis_name="s")
    k = pl.kernel(
        _emb_body,
        out_type=jax.ShapeDtypeStruct((S, 12, 32, 8, 128), jnp.float32),
        mesh=mesh,
        compiler_params=pltpu.CompilerParams(needs_layout_passes=False),
        scratch_types=[
            pltpu.VMEM((SLAB, 8, 128), jnp.int32),
            pltpu.VMEM((SLAB, 8, 128), jnp.int32),
            pltpu.VMEM((SLAB, 8, 128), jnp.int32),
            pltpu.VMEM((2, GRP, WPAD), jnp.float32),
            pltpu.VMEM((2, 12, 8, 128), jnp.float32),
            pltpu.VMEM((2, 8, 8, 128), jnp.float32),
            pltpu.VMEM((2, 8, 8, 128), jnp.float32),
            pltpu.SemaphoreType.DMA,
            pltpu.SemaphoreType.DMA,
            pltpu.SemaphoreType.DMA,
            pltpu.SemaphoreType.DMA,
        ],
    )
    wi = word.T.reshape(25, 8, 32, 128).transpose(0, 2, 1, 3)
    p1i = pos1.T.reshape(25, 8, 32, 128).transpose(0, 2, 1, 3)
    p2i = pos2.T.reshape(25, 8, 32, 128).transpose(0, 2, 1, 3)
    wt = jnp.pad(word_table, ((0, 0), (0, WPAD - WORD_SIZE)))
    p1t = jnp.pad(pos1_table.T, ((0, 0), (0, 24))).reshape(
        2, 8, 8, 128).transpose(0, 2, 1, 3)
    p2t = jnp.pad(pos2_table.T, ((0, 0), (0, 24))).reshape(
        2, 8, 8, 128).transpose(0, 2, 1, 3)
    out5 = k(wi, p1i, p2i, wt, p1t, p2t)
    return out5.transpose(2, 4, 0, 1, 3).reshape(B, S, OUT_SIZE)


def kernel(word, pos1, pos2, chars, word_table, pos1_table, pos2_table):
    del chars
    return _embed(word, pos1, pos2, word_table, pos1_table, pos2_table)


# speedup vs baseline: 2.4010x; 1.4267x over previous
import jax
import jax.numpy as jnp
from jax import lax
from jax.experimental import pallas as pl
from jax.experimental.pallas import tpu as pltpu
from jax.experimental.pallas import tpu_sc as plsc

B, S = 4096, 200
WORD_SIZE, POS_SIZE = 64, 16
OUT_SIZE = WORD_SIZE + 2 * POS_SIZE
WPAD = 128
N = B * S

NC, NS = 2, 16
NW = NC * NS
GRP = 128
NGRP = S
SLAB = 5
NBLK = NGRP // (8 * SLAB)


def _splat(v):
    return jnp.full((16,), v, jnp.int32)


def _emb_body(wi_hbm, p1i_hbm, p2i_hbm, wt_hbm, p1t_hbm, p2t_hbm, out_hbm,
              idxw_v, idx1_v, idx2_v, wbuf, tbuf, post1, post2,
              semG0, semG1, semW0, semW1):
    semG = (semG0, semG1)
    semW = (semW0, semW1)
    wid = lax.axis_index("s") * NC + lax.axis_index("c")
    iota16 = lax.iota(jnp.int32, 16)

    pltpu.sync_copy(p1t_hbm, post1)
    pltpu.sync_copy(p2t_hbm, post2)

    def fire_G(b, sbl, si):
        pltpu.async_copy(wt_hbm.at[idxw_v.at[sbl, si]], wbuf.at[b], semG[b])

    def wait_G(b):
        pltpu.make_async_copy(wt_hbm.at[idxw_v.at[0, 0]], wbuf.at[b],
                              semG[b]).wait()

    def fire_W(s, b):
        pltpu.async_copy(tbuf.at[b], out_hbm.at[s, :, wid], semW[b])

    def wait_W(b):
        pltpu.make_async_copy(tbuf.at[b], out_hbm.at[0, :, wid],
                              semW[b]).wait()

    def transpose_group(b, sbl, si):
        tb = tbuf.at[b]
        wb = wbuf.at[b]

        colv = [iota16 + 16 * c4 for c4 in range(4)]
        chi = [lax.shift_right_logical(cv, 3) for cv in colv]
        clo = [lax.bitwise_and(cv, 7) for cv in colv]

        @pl.loop(0, 16)
        def _(j):
            rb = lax.bitwise_and(iota16 + j, 15)
            for t in range(8):
                rowv = rb + 16 * t
                vs = [plsc.load_gather(wb, [rowv, colv[c4]])
                      for c4 in range(4)]
                for c4 in range(4):
                    plsc.store_scatter(tb, [chi[c4], clo[c4], rowv], vs[c4])

        @pl.loop(0, 8)
        def _(t):
            off = t * 16
            p1 = idx1_v[sbl, si, pl.ds(off, 16)]
            p2 = idx2_v[sbl, si, pl.ds(off, 16)]
            ph1 = lax.shift_right_logical(p1, 7)
            pl1 = lax.bitwise_and(p1, 127)
            ph2 = lax.shift_right_logical(p2, 7)
            pl2 = lax.bitwise_and(p2, 127)
            for c in range(POS_SIZE):
                v = plsc.load_gather(post1, [_splat(c // 8), ph1,
                                             _splat(c % 8), pl1])
                tb[8 + c // 8, c % 8, pl.ds(off, 16)] = v
            for c in range(POS_SIZE):
                v = plsc.load_gather(post2, [_splat(c // 8), ph2,
                                             _splat(c % 8), pl2])
                tb[10 + c // 8, c % 8, pl.ds(off, 16)] = v

    def step(k, gl, b, fire_next):
        wait_G(b)
        if fire_next:
            gl1 = gl + 1
            fire_G(1 - b, gl1 >> 3, gl1 & 7)
        g_total = 40 * k + gl

        @pl.when(g_total >= 2)
        def _():
            wait_W(b)

        transpose_group(b, gl >> 3, gl & 7)
        fire_W(g_total, b)

    @pl.loop(0, NBLK)
    def _(k):
        rows = pl.ds(SLAB * k, SLAB)
        pltpu.sync_copy(wi_hbm.at[rows, wid], idxw_v)
        pltpu.sync_copy(p1i_hbm.at[rows, wid], idx1_v)
        pltpu.sync_copy(p2i_hbm.at[rows, wid], idx2_v)
        fire_G(0, 0, 0)

        @pl.loop(0, 19)
        def _(j):
            step(k, 2 * j, 0, True)
            step(k, 2 * j + 1, 1, True)

        step(k, 38, 0, True)
        step(k, 39, 1, False)

    wait_W(0)
    wait_W(1)


@jax.jit
def _embed(word, pos1, pos2, word_table, pos1_table, pos2_table):
    mesh = plsc.VectorSubcoreMesh(core_axis_name="c", subcore_axis_name="s")
    k = pl.kernel(
        _emb_body,
        out_type=jax.ShapeDtypeStruct((S, 12, 32, 8, 128), jnp.float32),
        mesh=mesh,
        compiler_params=pltpu.CompilerParams(needs_layout_passes=False),
        scratch_types=[
            pltpu.VMEM((SLAB, 8, 128), jnp.int32),
            pltpu.VMEM((SLAB, 8, 128), jnp.int32),
            pltpu.VMEM((SLAB, 8, 128), jnp.int32),
            pltpu.VMEM((2, GRP, WPAD), jnp.float32),
            pltpu.VMEM((2, 12, 8, 128), jnp.float32),
            pltpu.VMEM((2, 8, 8, 128), jnp.float32),
            pltpu.VMEM((2, 8, 8, 128), jnp.float32),
            pltpu.SemaphoreType.DMA,
            pltpu.SemaphoreType.DMA,
            pltpu.SemaphoreType.DMA,
            pltpu.SemaphoreType.DMA,
        ],
    )
    wi = word.T.reshape(25, 8, 32, 128).transpose(0, 2, 1, 3)
    p1i = pos1.T.reshape(25, 8, 32, 128).transpose(0, 2, 1, 3)
    p2i = pos2.T.reshape(25, 8, 32, 128).transpose(0, 2, 1, 3)
    eye = jnp.eye(WORD_SIZE, WPAD, dtype=jnp.float32)
    wt = jax.lax.dot_general(word_table, eye, (((1,), (0,)), ((), ())),
                             precision=jax.lax.Precision.HIGHEST,
                             preferred_element_type=jnp.float32)
    p1t = jnp.pad(pos1_table.T, ((0, 0), (0, 24))).reshape(
        2, 8, 8, 128).transpose(0, 2, 1, 3)
    p2t = jnp.pad(pos2_table.T, ((0, 0), (0, 24))).reshape(
        2, 8, 8, 128).transpose(0, 2, 1, 3)
    out5 = k(wi, p1i, p2i, wt, p1t, p2t)
    return out5.transpose(2, 4, 0, 1, 3).reshape(B, S, OUT_SIZE)


def kernel(word, pos1, pos2, chars, word_table, pos1_table, pos2_table):
    del chars
    return _embed(word, pos1, pos2, word_table, pos1_table, pos2_table)


# speedup vs baseline: 2.6012x; 1.0834x over previous
import jax
import jax.numpy as jnp
from jax import lax
from jax.experimental import pallas as pl
from jax.experimental.pallas import tpu as pltpu
from jax.experimental.pallas import tpu_sc as plsc

B, S = 4096, 200
WORD_SIZE, POS_SIZE = 64, 16
OUT_SIZE = WORD_SIZE + 2 * POS_SIZE
WPAD = 128
N = B * S

NC, NS = 2, 16
NW = NC * NS
GRP = 128
NGRP = S
SLAB = 5
NBLK = NGRP // (8 * SLAB)


def _splat(v):
    return jnp.full((16,), v, jnp.int32)


def _emb_body(wi_hbm, p1i_hbm, p2i_hbm, wt_hbm, p1t_hbm, p2t_hbm, out_hbm,
              idxw_v, idx1_v, idx2_v, wbuf, tbuf, post1, post2,
              semG0, semG1, semW0, semW1):
    semG = (semG0, semG1)
    semW = (semW0, semW1)
    wid = lax.axis_index("s") * NC + lax.axis_index("c")
    iota16 = lax.iota(jnp.int32, 16)

    pltpu.sync_copy(p1t_hbm, post1)
    pltpu.sync_copy(p2t_hbm, post2)

    def fire_G(b, sbl, si):
        pltpu.async_copy(wt_hbm.at[idxw_v.at[sbl, si]], wbuf.at[b], semG[b])

    def wait_G(b):
        pltpu.make_async_copy(wt_hbm.at[idxw_v.at[0, 0]], wbuf.at[b],
                              semG[b]).wait()

    def fire_W(s, b):
        pltpu.async_copy(tbuf.at[b], out_hbm.at[s, :, wid], semW[b])

    def wait_W(b):
        pltpu.make_async_copy(tbuf.at[b], out_hbm.at[0, :, wid],
                              semW[b]).wait()

    def transpose_group(b, sbl, si):
        tb = tbuf.at[b]
        wb = wbuf.at[b]

        colv = [iota16 + 16 * c4 for c4 in range(4)]
        chi = [lax.shift_right_logical(cv, 3) for cv in colv]
        clo = [lax.bitwise_and(cv, 7) for cv in colv]

        @pl.loop(0, 16)
        def _(j):
            rb = lax.bitwise_and(iota16 + j, 15)
            for t2 in range(4):
                rows = [rb + 16 * (2 * t2), rb + 16 * (2 * t2 + 1)]
                vs = [plsc.load_gather(wb, [rowv, colv[c4]])
                      for rowv in rows for c4 in range(4)]
                i = 0
                for rowv in rows:
                    for c4 in range(4):
                        plsc.store_scatter(tb, [chi[c4], clo[c4], rowv],
                                           vs[i])
                        i += 1

        @pl.loop(0, 8)
        def _(t):
            off = t * 16
            p1 = idx1_v[sbl, si, pl.ds(off, 16)]
            p2 = idx2_v[sbl, si, pl.ds(off, 16)]
            ph1 = lax.shift_right_logical(p1, 7)
            pl1 = lax.bitwise_and(p1, 127)
            ph2 = lax.shift_right_logical(p2, 7)
            pl2 = lax.bitwise_and(p2, 127)
            for c8 in range(2):
                v1 = [plsc.load_gather(post1, [_splat(c8), ph1,
                                               _splat(c), pl1])
                      for c in range(8)]
                v2 = [plsc.load_gather(post2, [_splat(c8), ph2,
                                               _splat(c), pl2])
                      for c in range(8)]
                for c in range(8):
                    tb[8 + c8, c, pl.ds(off, 16)] = v1[c]
                    tb[10 + c8, c, pl.ds(off, 16)] = v2[c]

    def step(k, gl, b, fire_next):
        wait_G(b)
        if fire_next:
            gl1 = gl + 1
            fire_G(1 - b, gl1 >> 3, gl1 & 7)
        g_total = 40 * k + gl

        @pl.when(g_total >= 2)
        def _():
            wait_W(b)

        transpose_group(b, gl >> 3, gl & 7)
        fire_W(g_total, b)

    @pl.loop(0, NBLK)
    def _(k):
        rows = pl.ds(SLAB * k, SLAB)
        pltpu.sync_copy(wi_hbm.at[rows, wid], idxw_v)
        pltpu.sync_copy(p1i_hbm.at[rows, wid], idx1_v)
        pltpu.sync_copy(p2i_hbm.at[rows, wid], idx2_v)
        fire_G(0, 0, 0)

        @pl.loop(0, 19)
        def _(j):
            step(k, 2 * j, 0, True)
            step(k, 2 * j + 1, 1, True)

        step(k, 38, 0, True)
        step(k, 39, 1, False)

    wait_W(0)
    wait_W(1)


@jax.jit
def _embed(word, pos1, pos2, word_table, pos1_table, pos2_table):
    mesh = plsc.VectorSubcoreMesh(core_axis_name="c", subcore_axis_name="s")
    k = pl.kernel(
        _emb_body,
        out_type=jax.ShapeDtypeStruct((S, 12, 32, 8, 128), jnp.float32),
        mesh=mesh,
        compiler_params=pltpu.CompilerParams(needs_layout_passes=False),
        scratch_types=[
            pltpu.VMEM((SLAB, 8, 128), jnp.int32),
            pltpu.VMEM((SLAB, 8, 128), jnp.int32),
            pltpu.VMEM((SLAB, 8, 128), jnp.int32),
            pltpu.VMEM((2, GRP, WPAD), jnp.float32),
            pltpu.VMEM((2, 12, 8, 128), jnp.float32),
            pltpu.VMEM((2, 8, 8, 128), jnp.float32),
            pltpu.VMEM((2, 8, 8, 128), jnp.float32),
            pltpu.SemaphoreType.DMA,
            pltpu.SemaphoreType.DMA,
            pltpu.SemaphoreType.DMA,
            pltpu.SemaphoreType.DMA,
        ],
    )
    wi = word.T.reshape(25, 8, 32, 128).transpose(0, 2, 1, 3)
    p1i = pos1.T.reshape(25, 8, 32, 128).transpose(0, 2, 1, 3)
    p2i = pos2.T.reshape(25, 8, 32, 128).transpose(0, 2, 1, 3)
    eye = jnp.eye(WORD_SIZE, WPAD, dtype=jnp.float32)
    wt = jax.lax.dot_general(word_table, eye, (((1,), (0,)), ((), ())),
                             precision=jax.lax.Precision.HIGHEST,
                             preferred_element_type=jnp.float32)
    p1t = jnp.pad(pos1_table.T, ((0, 0), (0, 24))).reshape(
        2, 8, 8, 128).transpose(0, 2, 1, 3)
    p2t = jnp.pad(pos2_table.T, ((0, 0), (0, 24))).reshape(
        2, 8, 8, 128).transpose(0, 2, 1, 3)
    out5 = k(wi, p1i, p2i, wt, p1t, p2t)
    return out5.transpose(2, 4, 0, 1, 3).reshape(B, S, OUT_SIZE)


def kernel(word, pos1, pos2, chars, word_table, pos1_table, pos2_table):
    del chars
    return _embed(word, pos1, pos2, word_table, pos1_table, pos2_table)


# speedup vs baseline: 2.6118x; 1.0041x over previous
import jax
import jax.numpy as jnp
from jax import lax
from jax.experimental import pallas as pl
from jax.experimental.pallas import tpu as pltpu
from jax.experimental.pallas import tpu_sc as plsc

B, S = 4096, 200
WORD_SIZE, POS_SIZE = 64, 16
OUT_SIZE = WORD_SIZE + 2 * POS_SIZE
WPAD = 128
N = B * S

NC, NS = 2, 16
NW = NC * NS
GRP = 128
NGRP = S
SLAB = 5
NBLK = NGRP // (8 * SLAB)


def _splat(v):
    return jnp.full((16,), v, jnp.int32)


def _emb_body(wi_hbm, p1i_hbm, p2i_hbm, wt_hbm, p1t_hbm, p2t_hbm, out_hbm,
              idxw_v, idx1_v, idx2_v, wbuf, tbuf, post1, post2,
              semG0, semG1, semW0, semW1):
    semG = (semG0, semG1)
    semW = (semW0, semW1)
    wid = lax.axis_index("s") * NC + lax.axis_index("c")
    iota16 = lax.iota(jnp.int32, 16)

    pltpu.sync_copy(p1t_hbm, post1)
    pltpu.sync_copy(p2t_hbm, post2)

    def fire_G(b, sbl, si):
        pltpu.async_copy(wt_hbm.at[idxw_v.at[sbl, si]], wbuf.at[b], semG[b])

    def wait_G(b):
        pltpu.make_async_copy(wt_hbm.at[idxw_v.at[0, 0]], wbuf.at[b],
                              semG[b]).wait()

    def fire_W(s, b):
        pltpu.async_copy(tbuf.at[b], out_hbm.at[s, :, wid], semW[b])

    def wait_W(b):
        pltpu.make_async_copy(tbuf.at[b], out_hbm.at[0, :, wid],
                              semW[b]).wait()

    def transpose_group(b, sbl, si):
        tb = tbuf.at[b]
        wb = wbuf.at[b]

        colv = [iota16 + 16 * c4 for c4 in range(4)]
        chi = [lax.shift_right_logical(cv, 3) for cv in colv]
        clo = [lax.bitwise_and(cv, 7) for cv in colv]

        @pl.loop(0, 16)
        def _(j):
            rb = lax.bitwise_and(iota16 + j, 15)
            for t4 in range(2):
                rows = [rb + 16 * (4 * t4 + t) for t in range(4)]
                vs = [plsc.load_gather(wb, [rowv, colv[c4]])
                      for rowv in rows for c4 in range(4)]
                i = 0
                for rowv in rows:
                    for c4 in range(4):
                        plsc.store_scatter(tb, [chi[c4], clo[c4], rowv],
                                           vs[i])
                        i += 1

        @pl.loop(0, 8)
        def _(t):
            off = t * 16
            p1 = idx1_v[sbl, si, pl.ds(off, 16)]
            p2 = idx2_v[sbl, si, pl.ds(off, 16)]
            ph1 = lax.shift_right_logical(p1, 7)
            pl1 = lax.bitwise_and(p1, 127)
            ph2 = lax.shift_right_logical(p2, 7)
            pl2 = lax.bitwise_and(p2, 127)
            for c8 in range(2):
                v1 = [plsc.load_gather(post1, [_splat(c8), ph1,
                                               _splat(c), pl1])
                      for c in range(8)]
                v2 = [plsc.load_gather(post2, [_splat(c8), ph2,
                                               _splat(c), pl2])
                      for c in range(8)]
                for c in range(8):
                    tb[8 + c8, c, pl.ds(off, 16)] = v1[c]
                    tb[10 + c8, c, pl.ds(off, 16)] = v2[c]

    def step(k, gl, b, fire_next):
        wait_G(b)
        if fire_next:
            gl1 = gl + 1
            fire_G(1 - b, gl1 >> 3, gl1 & 7)
        g_total = 40 * k + gl

        @pl.when(g_total >= 2)
        def _():
            wait_W(b)

        transpose_group(b, gl >> 3, gl & 7)
        fire_W(g_total, b)

    @pl.loop(0, NBLK)
    def _(k):
        rows = pl.ds(SLAB * k, SLAB)
        pltpu.sync_copy(wi_hbm.at[rows, wid], idxw_v)
        pltpu.sync_copy(p1i_hbm.at[rows, wid], idx1_v)
        pltpu.sync_copy(p2i_hbm.at[rows, wid], idx2_v)
        fire_G(0, 0, 0)

        @pl.loop(0, 19)
        def _(j):
            step(k, 2 * j, 0, True)
            step(k, 2 * j + 1, 1, True)

        step(k, 38, 0, True)
        step(k, 39, 1, False)

    wait_W(0)
    wait_W(1)


@jax.jit
def _embed(word, pos1, pos2, word_table, pos1_table, pos2_table):
    mesh = plsc.VectorSubcoreMesh(core_axis_name="c", subcore_axis_name="s")
    k = pl.kernel(
        _emb_body,
        out_type=jax.ShapeDtypeStruct((S, 12, 32, 8, 128), jnp.float32),
        mesh=mesh,
        compiler_params=pltpu.CompilerParams(needs_layout_passes=False),
        scratch_types=[
            pltpu.VMEM((SLAB, 8, 128), jnp.int32),
            pltpu.VMEM((SLAB, 8, 128), jnp.int32),
            pltpu.VMEM((SLAB, 8, 128), jnp.int32),
            pltpu.VMEM((2, GRP, WPAD), jnp.float32),
            pltpu.VMEM((2, 12, 8, 128), jnp.float32),
            pltpu.VMEM((2, 8, 8, 128), jnp.float32),
            pltpu.VMEM((2, 8, 8, 128), jnp.float32),
            pltpu.SemaphoreType.DMA,
            pltpu.SemaphoreType.DMA,
            pltpu.SemaphoreType.DMA,
            pltpu.SemaphoreType.DMA,
        ],
    )
    wi = word.T.reshape(25, 8, 32, 128).transpose(0, 2, 1, 3)
    p1i = pos1.T.reshape(25, 8, 32, 128).transpose(0, 2, 1, 3)
    p2i = pos2.T.reshape(25, 8, 32, 128).transpose(0, 2, 1, 3)
    eye = jnp.eye(WORD_SIZE, WPAD, dtype=jnp.float32)
    wt = jax.lax.dot_general(word_table, eye, (((1,), (0,)), ((), ())),
                             precision=jax.lax.Precision.HIGHEST,
                             preferred_element_type=jnp.float32)
    p1t = jnp.pad(pos1_table.T, ((0, 0), (0, 24))).reshape(
        2, 8, 8, 128).transpose(0, 2, 1, 3)
    p2t = jnp.pad(pos2_table.T, ((0, 0), (0, 24))).reshape(
        2, 8, 8, 128).transpose(0, 2, 1, 3)
    out5 = k(wi, p1i, p2i, wt, p1t, p2t)
    return out5.transpose(2, 4, 0, 1, 3).reshape(B, S, OUT_SIZE)


def kernel(word, pos1, pos2, chars, word_table, pos1_table, pos2_table):
    del chars
    return _embed(word, pos1, pos2, word_table, pos1_table, pos2_table)
